# baseline (speedup 1.0000x reference)
import functools

import jax
import jax.numpy as jnp
from jax import lax
from jax.experimental import pallas as pl
from jax.experimental.pallas import tpu as pltpu
from jax.experimental.pallas import tpu_sc as plsc

_GAMMA = 12.0
_B = 16384
_D = 32
_NC = 2
_NS = 16
_NW = _NC * _NS
_BPW = _B // _NW
_CHUNK = 128
_NCHUNK = _BPW // _CHUNK
_L = 16
_SPAD = 17
_GRP = _BPW // _L

_mesh = plsc.VectorSubcoreMesh(
    core_axis_name="c", subcore_axis_name="s",
    num_cores=_NC, num_subcores=_NS)


@functools.partial(
    pl.kernel,
    out_type=jax.ShapeDtypeStruct((_NW, _BPW), jnp.float32),
    mesh=_mesh,
    compiler_params=pltpu.CompilerParams(
        needs_layout_passes=False, use_tc_tiling_on_sc=False),
    scratch_types=[
        pltpu.VMEM((_NCHUNK, _CHUNK), jnp.int32),
        pltpu.VMEM((_NCHUNK, _CHUNK), jnp.int32),
        pltpu.VMEM((_NCHUNK, _CHUNK), jnp.int32),
        pltpu.VMEM((_BPW, _D), jnp.float32),
        pltpu.VMEM((_BPW, _D), jnp.float32),
        pltpu.VMEM((_BPW, _D), jnp.float32),
        pltpu.VMEM((_BPW * _SPAD + _L,), jnp.float32),
        pltpu.VMEM((_BPW,), jnp.float32),
        pltpu.SemaphoreType.DMA,
    ],
)
def _kg_logits(ent, rel, hds, rls, tls, out,
               hidx, ridx, tidx, hv, rv, tv, sv, ov, sem):
    wid = lax.axis_index("s") * _NC + lax.axis_index("c")

    pltpu.sync_copy(hds.at[wid], hidx)
    pltpu.sync_copy(rls.at[wid], ridx)
    pltpu.sync_copy(tls.at[wid], tidx)

    copies = []
    for c in range(_NCHUNK):
        sl = pl.ds(c * _CHUNK, _CHUNK)
        copies.append(pltpu.async_copy(ent.at[hidx.at[c]], hv.at[sl], sem))
        copies.append(pltpu.async_copy(rel.at[ridx.at[c]], rv.at[sl], sem))
        copies.append(pltpu.async_copy(ent.at[tidx.at[c]], tv.at[sl], sem))
    for cp in copies:
        cp.wait()

    iota = lax.iota(jnp.int32, _L)

    def row_body(r, carry):
        h0 = hv[r, pl.ds(0, _L)]
        h1 = hv[r, pl.ds(_L, _L)]
        r0 = rv[r, pl.ds(0, _L)]
        r1 = rv[r, pl.ds(_L, _L)]
        t0 = tv[r, pl.ds(0, _L)]
        t1 = tv[r, pl.ds(_L, _L)]
        s = jnp.abs(h0 + r0 - t0) + jnp.abs(h1 + r1 - t1)
        plsc.store_scatter(sv, [r * _SPAD + iota], s)
        return carry

    lax.fori_loop(0, _BPW, row_body, 0)

    def grp_body(g, carry):
        base = g * (_L * _SPAD)
        acc = jnp.zeros((_L,), jnp.float32)
        for j in range(_L):
            acc = acc + plsc.load_gather(sv, [base + j + iota * _SPAD])
        ov[pl.ds(g * _L, _L)] = _GAMMA - acc
        return carry

    lax.fori_loop(0, _GRP, grp_body, 0)

    pltpu.sync_copy(ov, out.at[wid])


def kernel(entity_embedding, relation_embedding, heads, relations, tails):
    hds = heads.astype(jnp.int32).reshape(_NW, _NCHUNK, _CHUNK)
    rls = relations.astype(jnp.int32).reshape(_NW, _NCHUNK, _CHUNK)
    tls = tails.astype(jnp.int32).reshape(_NW, _NCHUNK, _CHUNK)
    out = _kg_logits(entity_embedding, relation_embedding, hds, rls, tls)
    return out.reshape(_B)
